# baseline (speedup 1.0000x reference)
import functools

import jax
import jax.numpy as jnp
from jax import lax
from jax.experimental import pallas as pl
from jax.experimental.pallas import tpu as pltpu
from jax.experimental.pallas import tpu_sc as plsc

M = 1024
C = 128
KNN = 3

NUM_CORES = 2
NUM_SUBCORES = 16
NW = NUM_CORES * NUM_SUBCORES

CH = 104
NCHUNK = 16
ROWS_PER_W = CH * NCHUNK
NPAD = NW * ROWS_PER_W

RBLK = 1024


def _topk_body(v_ref, ct_ref, cf_ref, w1_ref, b_ref, idx_ref, w_ref, g_ref):
    @pl.when(pl.program_id(0) == 0)
    def _table():
        g_ref[...] = b_ref[...] + lax.dot_general(
            cf_ref[...], w1_ref[...], (((1,), (1,)), ((), ())),
            preferred_element_type=jnp.float32)

    d2 = None
    for j in range(3):
        t = v_ref[:, j:j + 1] - ct_ref[j:j + 1, :]
        d2 = t * t if d2 is None else d2 + t * t
    iota = lax.broadcasted_iota(jnp.int32, d2.shape, 1).astype(jnp.float32)
    big = jnp.float32(3e9)
    inf = jnp.float32(jnp.inf)
    idxs, d2s = [], []
    cur = d2
    for k in range(KNN):
        m = jnp.min(cur, axis=1, keepdims=True)
        eq = cur == m
        selx = jnp.min(jnp.where(eq, iota, big), axis=1, keepdims=True)
        idxs.append(selx.astype(jnp.int32))
        d2s.append(m)
        if k < KNN - 1:
            cur = jnp.where(eq, inf, cur)
    nz = d2s[0] != 0.0
    ws = []
    for dd in d2s:
        safe = jnp.where(dd == 0.0, 1.0, dd)
        inv = 1.0 / safe
        ws.append(jnp.where(nz, inv, jnp.where(dd == 0.0, 1.0, 0.0)))
    wsum = ws[0] + ws[1] + ws[2]
    idx_ref[...] = jnp.concatenate(idxs, axis=1)
    w_ref[...] = jnp.concatenate(
        [jnp.broadcast_to(w / wsum, (RBLK, 16)) for w in ws], axis=1)


def _interp_body(g_hbm, idxf_hbm, wf_hbm, out_hbm,
                 idx_v, w_v, rows_v, out_v, sem0, sem1):
    wid = lax.axis_index("s") * NUM_CORES + lax.axis_index("c")
    base = wid * ROWS_PER_W
    sems = [sem0, sem1]

    for j in range(KNN):
        off = pl.multiple_of(j * NPAD + base, 8)
        dst = pl.ds(j * ROWS_PER_W, ROWS_PER_W)
        pltpu.sync_copy(idxf_hbm.at[pl.ds(off, ROWS_PER_W)], idx_v.at[dst])

    def fire(cidx, b):
        r0 = cidx * CH
        for j in range(KNN):
            src = pl.ds(pl.multiple_of(j * ROWS_PER_W + r0, 8), CH)
            pltpu.async_copy(g_hbm.at[idx_v.at[src]],
                             rows_v.at[b, j], sems[b])
        woff = pl.multiple_of((base + r0) * (KNN * 16), 8)
        pltpu.async_copy(wf_hbm.at[pl.ds(woff, CH * KNN * 16)],
                         w_v.at[b], sems[b])

    def drain(b):
        for j in range(KNN):
            pltpu.make_async_copy(g_hbm.at[pl.ds(0, CH), :],
                                  rows_v.at[b, j], sems[b]).wait()
        pltpu.make_async_copy(wf_hbm.at[pl.ds(0, CH * KNN * 16)],
                              w_v.at[b], sems[b]).wait()

    def compute(cidx, b):
        def row_body(i, c2):
            wrow = i * (KNN * 16)
            w0 = w_v[b, pl.ds(wrow, 16)]
            w1 = w_v[b, pl.ds(wrow + 16, 16)]
            w2 = w_v[b, pl.ds(wrow + 32, 16)]
            for l in range(C // 16):
                s = pl.ds(l * 16, 16)
                out_v[b, i, s] = (w0 * rows_v[b, 0, i, s]
                                  + w1 * rows_v[b, 1, i, s]
                                  + w2 * rows_v[b, 2, i, s])
            return c2

        lax.fori_loop(0, CH, row_body, 0)
        rowc = pl.multiple_of(base + cidx * CH, 8)
        pltpu.sync_copy(out_v.at[b], out_hbm.at[pl.ds(rowc, CH), :])

    fire(0, 0)

    def pair_body(p, carry):
        for b in range(2):
            c = 2 * p + b
            fire(jnp.minimum(c + 1, NCHUNK - 1), 1 - b)
            drain(b)
            compute(c, b)
        return carry

    lax.fori_loop(0, NCHUNK // 2 - 1, pair_body, 0)
    c = NCHUNK - 2
    fire(c + 1, 1)
    drain(0)
    compute(c, 0)
    fire(c + 1, 0)
    drain(1)
    compute(c + 1, 1)
    drain(0)


def _make_topk():
    grid = (NPAD // RBLK,)
    return pl.pallas_call(
        _topk_body,
        grid=grid,
        in_specs=[
            pl.BlockSpec((RBLK, 8), lambda i: (i, 0)),
            pl.BlockSpec((8, M), lambda i: (0, 0)),
            pl.BlockSpec((M, C), lambda i: (0, 0)),
            pl.BlockSpec((C, C), lambda i: (0, 0)),
            pl.BlockSpec((1, C), lambda i: (0, 0)),
        ],
        out_specs=[
            pl.BlockSpec((RBLK, KNN), lambda i: (i, 0)),
            pl.BlockSpec((RBLK, KNN * 16), lambda i: (i, 0)),
            pl.BlockSpec((M, C), lambda i: (0, 0)),
        ],
        out_shape=[
            jax.ShapeDtypeStruct((NPAD, KNN), jnp.int32),
            jax.ShapeDtypeStruct((NPAD, KNN * 16), jnp.float32),
            jax.ShapeDtypeStruct((M, C), jnp.float32),
        ],
    )


def _make_interp():
    mesh = plsc.VectorSubcoreMesh(
        core_axis_name="c", subcore_axis_name="s",
        num_cores=NUM_CORES, num_subcores=NUM_SUBCORES)
    return functools.partial(
        pl.kernel,
        out_type=jax.ShapeDtypeStruct((NPAD, C), jnp.float32),
        mesh=mesh,
        scratch_types=[
            pltpu.VMEM((KNN * ROWS_PER_W,), jnp.int32),
            pltpu.VMEM((2, CH * KNN * 16), jnp.float32),
            pltpu.VMEM((2, KNN, CH, C), jnp.float32),
            pltpu.VMEM((2, CH, C), jnp.float32),
            pltpu.SemaphoreType.DMA,
            pltpu.SemaphoreType.DMA,
        ],
    )(_interp_body)


@jax.jit
def kernel(vertices, centroids, centroid_features, W, b):
    n = vertices.shape[0]
    vpad = jnp.zeros((NPAD, 8), jnp.float32).at[:n, :3].set(vertices)
    ct8 = jnp.zeros((8, M), jnp.float32).at[:3].set(centroids.T)
    idx, w, g = _make_topk()(vpad, ct8, centroid_features, W[:, :C],
                             b.reshape(1, C))
    out = _make_interp()(g, idx.T.reshape(-1), w.reshape(-1))
    return out[:n]


# speedup vs baseline: 1.0944x; 1.0944x over previous
import functools

import jax
import jax.numpy as jnp
from jax import lax
from jax.experimental import pallas as pl
from jax.experimental.pallas import tpu as pltpu
from jax.experimental.pallas import tpu_sc as plsc

M = 1024
C = 128
KNN = 3

NUM_CORES = 2
NUM_SUBCORES = 16
NW = NUM_CORES * NUM_SUBCORES

CH = 48
NCHUNK = 34
ROWS_PER_W = CH * NCHUNK
NPAD = NW * ROWS_PER_W

RBLK = 1024


def _topk_body(v_ref, ct_ref, cf_ref, w1_ref, b_ref, idx_ref, w_ref, g_ref):
    @pl.when(pl.program_id(0) == 0)
    def _table():
        g_ref[...] = b_ref[...] + lax.dot_general(
            cf_ref[...], w1_ref[...], (((1,), (1,)), ((), ())),
            preferred_element_type=jnp.float32)

    d2 = None
    for j in range(3):
        t = v_ref[:, j:j + 1] - ct_ref[j:j + 1, :]
        d2 = t * t if d2 is None else d2 + t * t
    iota = lax.broadcasted_iota(jnp.int32, d2.shape, 1).astype(jnp.float32)
    big = jnp.float32(3e9)
    inf = jnp.float32(jnp.inf)
    idxs, d2s = [], []
    cur = d2
    for k in range(KNN):
        m = jnp.min(cur, axis=1, keepdims=True)
        eq = cur == m
        selx = jnp.min(jnp.where(eq, iota, big), axis=1, keepdims=True)
        idxs.append(selx.astype(jnp.int32))
        d2s.append(m)
        if k < KNN - 1:
            cur = jnp.where(eq, inf, cur)
    nz = d2s[0] != 0.0
    ws = []
    for dd in d2s:
        safe = jnp.where(dd == 0.0, 1.0, dd)
        inv = 1.0 / safe
        ws.append(jnp.where(nz, inv, jnp.where(dd == 0.0, 1.0, 0.0)))
    wsum = ws[0] + ws[1] + ws[2]
    idx_ref[...] = jnp.concatenate(idxs, axis=1)
    w_ref[...] = jnp.concatenate(
        [jnp.broadcast_to(w / wsum, (RBLK, 16)) for w in ws], axis=1)


def _interp_body(g_hbm, idxf_hbm, wf_hbm, out_hbm,
                 idx_v, w_v, rows_v, out_v, sem0, sem1):
    wid = lax.axis_index("s") * NUM_CORES + lax.axis_index("c")
    base = wid * ROWS_PER_W
    sems = [sem0, sem1]

    for j in range(KNN):
        off = pl.multiple_of(j * NPAD + base, 8)
        dst = pl.ds(j * ROWS_PER_W, ROWS_PER_W)
        pltpu.sync_copy(idxf_hbm.at[pl.ds(off, ROWS_PER_W)], idx_v.at[dst])

    def fire(cidx, b):
        r0 = cidx * CH
        for j in range(KNN):
            src = pl.ds(pl.multiple_of(j * ROWS_PER_W + r0, 8), CH)
            pltpu.async_copy(g_hbm.at[idx_v.at[src]],
                             rows_v.at[b, j], sems[b])
        woff = pl.multiple_of((base + r0) * (KNN * 16), 8)
        pltpu.async_copy(wf_hbm.at[pl.ds(woff, CH * KNN * 16)],
                         w_v.at[b], sems[b])

    def drain(b):
        for j in range(KNN):
            pltpu.make_async_copy(g_hbm.at[pl.ds(0, CH), :],
                                  rows_v.at[b, j], sems[b]).wait()
        pltpu.make_async_copy(wf_hbm.at[pl.ds(0, CH * KNN * 16)],
                              w_v.at[b], sems[b]).wait()

    def compute(cidx, b):
        def row_body(i, c2):
            wrow = i * (KNN * 16)
            w0 = w_v[b, pl.ds(wrow, 16)]
            w1 = w_v[b, pl.ds(wrow + 16, 16)]
            w2 = w_v[b, pl.ds(wrow + 32, 16)]
            for l in range(C // 16):
                s = pl.ds(l * 16, 16)
                out_v[b, i, s] = (w0 * rows_v[b, 0, i, s]
                                  + w1 * rows_v[b, 1, i, s]
                                  + w2 * rows_v[b, 2, i, s])
            return c2

        lax.fori_loop(0, CH, row_body, 0)
        rowc = pl.multiple_of(base + cidx * CH, 8)
        pltpu.sync_copy(out_v.at[b], out_hbm.at[pl.ds(rowc, CH), :])

    fire(0, 0)

    def pair_body(p, carry):
        for b in range(2):
            c = 2 * p + b
            fire(jnp.minimum(c + 1, NCHUNK - 1), 1 - b)
            drain(b)
            compute(c, b)
        return carry

    lax.fori_loop(0, NCHUNK // 2 - 1, pair_body, 0)
    c = NCHUNK - 2
    fire(c + 1, 1)
    drain(0)
    compute(c, 0)
    fire(c + 1, 0)
    drain(1)
    compute(c + 1, 1)
    drain(0)


def _make_topk():
    grid = (NPAD // RBLK,)
    return pl.pallas_call(
        _topk_body,
        grid=grid,
        in_specs=[
            pl.BlockSpec((RBLK, 8), lambda i: (i, 0)),
            pl.BlockSpec((8, M), lambda i: (0, 0)),
            pl.BlockSpec((M, C), lambda i: (0, 0)),
            pl.BlockSpec((C, C), lambda i: (0, 0)),
            pl.BlockSpec((1, C), lambda i: (0, 0)),
        ],
        out_specs=[
            pl.BlockSpec((RBLK, KNN), lambda i: (i, 0)),
            pl.BlockSpec((RBLK, KNN * 16), lambda i: (i, 0)),
            pl.BlockSpec((M, C), lambda i: (0, 0)),
        ],
        out_shape=[
            jax.ShapeDtypeStruct((NPAD, KNN), jnp.int32),
            jax.ShapeDtypeStruct((NPAD, KNN * 16), jnp.float32),
            jax.ShapeDtypeStruct((M, C), jnp.float32),
        ],
    )


def _make_interp():
    mesh = plsc.VectorSubcoreMesh(
        core_axis_name="c", subcore_axis_name="s",
        num_cores=NUM_CORES, num_subcores=NUM_SUBCORES)
    return functools.partial(
        pl.kernel,
        out_type=jax.ShapeDtypeStruct((NPAD, C), jnp.float32),
        mesh=mesh,
        scratch_types=[
            pltpu.VMEM((KNN * ROWS_PER_W,), jnp.int32),
            pltpu.VMEM((2, CH * KNN * 16), jnp.float32),
            pltpu.VMEM((2, KNN, CH, C), jnp.float32),
            pltpu.VMEM((2, CH, C), jnp.float32),
            pltpu.SemaphoreType.DMA,
            pltpu.SemaphoreType.DMA,
        ],
    )(_interp_body)


@jax.jit
def kernel(vertices, centroids, centroid_features, W, b):
    n = vertices.shape[0]
    vpad = jnp.zeros((NPAD, 8), jnp.float32).at[:n, :3].set(vertices)
    ct8 = jnp.zeros((8, M), jnp.float32).at[:3].set(centroids.T)
    idx, w, g = _make_topk()(vpad, ct8, centroid_features, W[:, :C],
                             b.reshape(1, C))
    out = _make_interp()(g, idx.T.reshape(-1), w.reshape(-1))
    return out[:n]
